# baseline (speedup 1.0000x reference)
import jax
import jax.numpy as jnp
from jax import lax
from jax.experimental import pallas as pl
from jax.experimental.pallas import tpu as pltpu
from jax.experimental.pallas import tpu_sc as plsc

N_NODES = 10000
N_EDGES = 160000
HID = 256
POS_VOCAB = 10000

NC = 2
NS = 16
NW = NC * NS
L = 16

C = 320
NP = NW * C
RPS = 16
NSTR = C // RPS
SUB = 80
NSUB = C // SUB
PWIN = SUB + 8

BE = 16000
NB = N_EDGES // BE


def _node_body(ids_hbm, mask_hbm, ntab_hbm, ptab_hbm, out_hbm,
               ids_v, mask_v, rank_v, nbufs, pbuf,
               sems_n, sem_p, sems_w, sem_s):
    cid = lax.axis_index("c")
    sid = lax.axis_index("s")
    wid = sid * NC + cid
    base = wid * C

    SPS = SUB // RPS
    with jax.named_scope("nk_stage"):
        pltpu.async_copy(ids_hbm.at[pl.ds(base, C)], ids_v, sem_s).wait()
        cp_mask = pltpu.async_copy(mask_hbm, mask_v.at[pl.ds(0, NP)], sem_s)
    gn = [pltpu.async_copy(ntab_hbm.at[ids_v.at[pl.ds(j * RPS, RPS)]],
                           nbufs[j // SPS].at[pl.ds((j % SPS) * RPS, RPS)],
                           sems_n[j // SPS])
          for j in range(NSTR)]
    cp_mask.wait()

    with jax.named_scope("nk_prefix"):
        def pre_body(k, acc):
            m = mask_v[pl.ds(k * L, L)]
            return acc + jnp.where(m == 1, 1, 0)

        accv = lax.fori_loop(0, base // L, pre_body, jnp.zeros((L,), jnp.int32))
        prefix = jnp.sum(accv)

    with jax.named_scope("nk_ranks"):
        starts = []
        carry = jnp.zeros((), jnp.int32)
        for k in range(C // L):
            if k % (SUB // L) == 0:
                st = jnp.minimum(prefix + carry, POS_VOCAB - PWIN)
                starts.append(pl.multiple_of((st // 8) * 8, 8))
            m = mask_v[pl.ds(base + k * L, L)]
            mi = jnp.where(m == 1, 1, 0)
            cs = plsc.cumsum(mi)
            rank_v[pl.ds(k * L, L)] = prefix + carry + (cs - mi)
            carry = carry + jnp.sum(mi)

    gp = pltpu.async_copy(ptab_hbm.at[pl.ds(starts[0], PWIN)], pbuf, sem_p)

    for s in range(NSUB):
        with jax.named_scope("nk_wait"):
            for j in range(SPS):
                gn[SPS * s + j].wait()
            gp.wait()
        with jax.named_scope("nk_add"):
            def add_body(j, _, nb=nbufs[s], st=starts[s], s=s):
                m = mask_v[pl.ds(base + s * SUB + j, L)][0]

                @pl.when(m == 1)
                def _masked_add():
                    rel = rank_v[pl.ds(s * SUB + j, L)][0] - st
                    for cb in range(HID // L):
                        nb[j, pl.ds(cb * L, L)] = (
                            nb[j, pl.ds(cb * L, L)]
                            + pbuf[rel, pl.ds(cb * L, L)])

                return 0

            lax.fori_loop(0, SUB, add_body, 0)
        @pl.when(base + s * SUB < N_NODES)
        def _write(s=s):
            pltpu.async_copy(nbufs[s], out_hbm.at[pl.ds(base + s * SUB, SUB)],
                             sems_w[s])

        if s + 1 < NSUB:
            gp = pltpu.async_copy(ptab_hbm.at[pl.ds(starts[s + 1], PWIN)],
                                  pbuf, sem_p)
    with jax.named_scope("nk_drain"):
        for s in range(NSUB):
            @pl.when(base + s * SUB < N_NODES)
            def _drain(s=s):
                pltpu.make_async_copy(
                    nbufs[s], out_hbm.at[pl.ds(base + s * SUB, SUB)],
                    sems_w[s]).wait()


def _edge_body(ids_ref, tbl_ref, g_ref, b_ref, out_ref):
    tbl = tbl_ref[...]
    mean = jnp.mean(tbl, axis=1, keepdims=True)
    var = jnp.mean((tbl - mean) ** 2, axis=1, keepdims=True)
    norm = (tbl - mean) * lax.rsqrt(var + 1e-5) * g_ref[...] + b_ref[...]
    norm_hi = norm.astype(jnp.bfloat16)
    norm_lo = (norm - norm_hi.astype(jnp.float32)).astype(jnp.bfloat16)
    ids = ids_ref[0, 0, :]
    onehot = (ids[:, None] == lax.broadcasted_iota(jnp.int32, (BE, 64), 1)
              ).astype(jnp.bfloat16)
    out_ref[...] = (
        jnp.dot(onehot, norm_hi, preferred_element_type=jnp.float32)
        + jnp.dot(onehot, norm_lo, preferred_element_type=jnp.float32))


@jax.jit
def kernel(node_ids, top_mask, edge_ids, node_table, pos_table, edge_table,
           ln_gamma, ln_beta):
    node_ids = node_ids.astype(jnp.int32)
    ids_pad = jnp.pad(node_ids, (0, NP - N_NODES))
    mask_pad = jnp.pad(top_mask.astype(jnp.int32), (0, NP - N_NODES))

    node_feat = pl.kernel(
        _node_body,
        out_type=jax.ShapeDtypeStruct((N_NODES, HID), jnp.float32),
        mesh=plsc.VectorSubcoreMesh(core_axis_name="c", subcore_axis_name="s",
                                    num_cores=NC, num_subcores=NS),
        scratch_types=[
            pltpu.VMEM((C,), jnp.int32),
            pltpu.VMEM((NP + L,), jnp.int32),
            pltpu.VMEM((C + L,), jnp.int32),
            [pltpu.VMEM((SUB, HID), jnp.float32)] * NSUB,
            pltpu.VMEM((PWIN, HID), jnp.float32),
            [pltpu.SemaphoreType.DMA] * NSUB,
            pltpu.SemaphoreType.DMA,
            [pltpu.SemaphoreType.DMA] * NSUB,
            pltpu.SemaphoreType.DMA,
        ],
        compiler_params=pltpu.CompilerParams(needs_layout_passes=False),
    )(ids_pad, mask_pad, node_table, pos_table)

    edge_feat = pl.pallas_call(
        _edge_body,
        grid=(NB,),
        in_specs=[
            pl.BlockSpec((1, 1, BE), lambda i: (i, 0, 0)),
            pl.BlockSpec((64, HID), lambda i: (0, 0)),
            pl.BlockSpec((1, HID), lambda i: (0, 0)),
            pl.BlockSpec((1, HID), lambda i: (0, 0)),
        ],
        out_specs=pl.BlockSpec((BE, HID), lambda i: (i, 0)),
        out_shape=jax.ShapeDtypeStruct((N_EDGES, HID), jnp.float32),
    )(edge_ids.reshape(NB, 1, BE), edge_table,
      ln_gamma.reshape(1, HID), ln_beta.reshape(1, HID))

    return node_feat, edge_feat


# speedup vs baseline: 1.3331x; 1.3331x over previous
import jax
import jax.numpy as jnp
from jax import lax
from jax.experimental import pallas as pl
from jax.experimental.pallas import tpu as pltpu
from jax.experimental.pallas import tpu_sc as plsc

N_NODES = 10000
N_EDGES = 160000
HID = 256
POS_VOCAB = 10000

NC = 2
NS = 16
NW = NC * NS
L = 16

C = 320
NP = NW * C
RPS = 16
NSTR = C // RPS
SUB = 80
NSUB = C // SUB
PWIN = SUB + 8

BE = 16000
NB = N_EDGES // BE


def _node_body(ids_hbm, mask_hbm, ntab_hbm, ptab_hbm, out_hbm,
               ids_v, mask_v, crow_v, nbufs, pbuf,
               sems_n, sem_p, sems_w, sem_s):
    cid = lax.axis_index("c")
    sid = lax.axis_index("s")
    wid = sid * NC + cid
    base = pl.multiple_of(jnp.minimum(wid * C, N_NODES - C), 8)

    SPS = SUB // RPS
    with jax.named_scope("nk_stage"):
        pltpu.async_copy(ids_hbm.at[pl.ds(base, C)], ids_v, sem_s).wait()
        cp_mask = pltpu.async_copy(mask_hbm, mask_v.at[pl.ds(0, N_NODES)],
                                   sem_s)
    gn = [pltpu.async_copy(ntab_hbm.at[ids_v.at[pl.ds(j * RPS, RPS)]],
                           nbufs[j // SPS].at[pl.ds((j % SPS) * RPS, RPS)],
                           sems_n[j // SPS])
          for j in range(NSTR)]
    cp_mask.wait()

    with jax.named_scope("nk_prefix"):
        def pre_body(k, acc):
            m = mask_v[pl.ds(k * L, L)]
            return acc + jnp.where(m == 1, 1, 0)

        accv = lax.fori_loop(0, base // L, pre_body, jnp.zeros((L,), jnp.int32))
        prefix = jnp.sum(accv)

    with jax.named_scope("nk_ranks"):
        starts, dels, counts = [], [], []
        lane = lax.iota(jnp.int32, L)
        carry = jnp.zeros((), jnp.int32)
        cnt_sub = None
        for k in range(C // L):
            kk = k % (SUB // L)
            s = k // (SUB // L)
            if kk == 0:
                st = jnp.minimum(prefix + carry, POS_VOCAB - PWIN)
                st = pl.multiple_of((st // 8) * 8, 8)
                starts.append(st)
                dels.append(prefix + carry - st)
                cnt_sub = jnp.zeros((), jnp.int32)
            m = mask_v[pl.ds(base + k * L, L)]
            mq = m == 1
            plsc.store_compressed(crow_v.at[pl.ds(s * SUB + cnt_sub, L)],
                                  kk * L + lane, mask=mq)
            nm = jnp.sum(jnp.where(mq, 1, 0))
            cnt_sub = cnt_sub + nm
            carry = carry + nm
            if kk == SUB // L - 1:
                counts.append(cnt_sub)

    gp = pltpu.async_copy(ptab_hbm.at[pl.ds(starts[0], PWIN)], pbuf, sem_p)

    for s in range(NSUB):
        with jax.named_scope("nk_wait"):
            for j in range(SPS):
                gn[SPS * s + j].wait()
            gp.wait()
        with jax.named_scope("nk_add"):
            def add_body(i, _, nb=nbufs[s], d=dels[s], s=s):
                j = crow_v[pl.ds(s * SUB + i, L)][0]
                rel = d + i
                for cb in range(HID // L):
                    nb[j, pl.ds(cb * L, L)] = (
                        nb[j, pl.ds(cb * L, L)]
                        + pbuf[rel, pl.ds(cb * L, L)])
                return 0

            lax.fori_loop(0, counts[s], add_body, 0)
        wr = pltpu.async_copy(nbufs[s], out_hbm.at[pl.ds(base + s * SUB, SUB)],
                              sems_w[s])
        if s + 1 < NSUB:
            gp = pltpu.async_copy(ptab_hbm.at[pl.ds(starts[s + 1], PWIN)],
                                  pbuf, sem_p)
    with jax.named_scope("nk_drain"):
        for s in range(NSUB):
            pltpu.make_async_copy(
                nbufs[s], out_hbm.at[pl.ds(base + s * SUB, SUB)],
                sems_w[s]).wait()


def _edge_body(ids_ref, tbl_ref, g_ref, b_ref, out_ref):
    tbl = tbl_ref[...]
    mean = jnp.mean(tbl, axis=1, keepdims=True)
    var = jnp.mean((tbl - mean) ** 2, axis=1, keepdims=True)
    norm = (tbl - mean) * lax.rsqrt(var + 1e-5) * g_ref[...] + b_ref[...]
    norm_hi = norm.astype(jnp.bfloat16)
    norm_lo = (norm - norm_hi.astype(jnp.float32)).astype(jnp.bfloat16)
    ids = ids_ref[0, 0, :]
    onehot = (ids[:, None] == lax.broadcasted_iota(jnp.int32, (BE, 64), 1)
              ).astype(jnp.bfloat16)
    out_ref[...] = (
        jnp.dot(onehot, norm_hi, preferred_element_type=jnp.float32)
        + jnp.dot(onehot, norm_lo, preferred_element_type=jnp.float32))


@jax.jit
def kernel(node_ids, top_mask, edge_ids, node_table, pos_table, edge_table,
           ln_gamma, ln_beta):
    node_ids = node_ids.astype(jnp.int32)
    top_mask = top_mask.astype(jnp.int32)

    node_feat = pl.kernel(
        _node_body,
        out_type=jax.ShapeDtypeStruct((N_NODES, HID), jnp.float32),
        mesh=plsc.VectorSubcoreMesh(core_axis_name="c", subcore_axis_name="s",
                                    num_cores=NC, num_subcores=NS),
        scratch_types=[
            pltpu.VMEM((C,), jnp.int32),
            pltpu.VMEM((N_NODES + L,), jnp.int32),
            pltpu.VMEM((C + L,), jnp.int32),
            [pltpu.VMEM((SUB, HID), jnp.float32)] * NSUB,
            pltpu.VMEM((PWIN, HID), jnp.float32),
            [pltpu.SemaphoreType.DMA] * NSUB,
            pltpu.SemaphoreType.DMA,
            [pltpu.SemaphoreType.DMA] * NSUB,
            pltpu.SemaphoreType.DMA,
        ],
        compiler_params=pltpu.CompilerParams(needs_layout_passes=False),
    )(node_ids, top_mask, node_table, pos_table)

    edge_feat = pl.pallas_call(
        _edge_body,
        grid=(NB,),
        in_specs=[
            pl.BlockSpec((1, 1, BE), lambda i: (i, 0, 0)),
            pl.BlockSpec((64, HID), lambda i: (0, 0)),
            pl.BlockSpec((1, HID), lambda i: (0, 0)),
            pl.BlockSpec((1, HID), lambda i: (0, 0)),
        ],
        out_specs=pl.BlockSpec((BE, HID), lambda i: (i, 0)),
        out_shape=jax.ShapeDtypeStruct((N_EDGES, HID), jnp.float32),
    )(edge_ids.reshape(NB, 1, BE), edge_table,
      ln_gamma.reshape(1, HID), ln_beta.reshape(1, HID))

    return node_feat, edge_feat


# speedup vs baseline: 1.3387x; 1.0042x over previous
import jax
import jax.numpy as jnp
from jax import lax
from jax.experimental import pallas as pl
from jax.experimental.pallas import tpu as pltpu
from jax.experimental.pallas import tpu_sc as plsc

N_NODES = 10000
N_EDGES = 160000
HID = 256
POS_VOCAB = 10000

NC = 2
NS = 16
NW = NC * NS
L = 16

C = 320
RPS = 16
NSTR = C // RPS
SUB = 80
NSUB = C // SUB
PWIN = SUB + 8

BE = 16000
NB = N_EDGES // BE


def _node_body(ids_hbm, mask_hbm, ntab_hbm, ptab_hbm, out_hbm,
               ids_v, mask_v, crow_v, nbufs, pbuf,
               sems_n, sem_p, sems_w, sem_s):
    cid = lax.axis_index("c")
    sid = lax.axis_index("s")
    wid = sid * NC + cid
    base = pl.multiple_of(jnp.minimum(wid * C, N_NODES - C), 8)

    SPS = SUB // RPS
    with jax.named_scope("nk_stage"):
        pltpu.async_copy(ids_hbm.at[pl.ds(base, C)], ids_v, sem_s).wait()
        cp_mask = pltpu.async_copy(mask_hbm, mask_v.at[pl.ds(0, N_NODES)],
                                   sem_s)
    gn = [pltpu.async_copy(ntab_hbm.at[ids_v.at[pl.ds(j * RPS, RPS)]],
                           nbufs[j // SPS].at[pl.ds((j % SPS) * RPS, RPS)],
                           sems_n[j // SPS])
          for j in range(NSTR)]
    cp_mask.wait()

    with jax.named_scope("nk_prefix"):
        def pre_body(k, acc):
            m = mask_v[pl.ds(k * L, L)]
            return acc + jnp.where(m == 1, 1, 0)

        accv = lax.fori_loop(0, base // L, pre_body, jnp.zeros((L,), jnp.int32))
        prefix = jnp.sum(accv)

    with jax.named_scope("nk_ranks"):
        starts, dels, counts = [], [], []
        lane = lax.iota(jnp.int32, L)
        carry = jnp.zeros((), jnp.int32)
        cnt_sub = None
        for k in range(C // L):
            kk = k % (SUB // L)
            s = k // (SUB // L)
            if kk == 0:
                st = jnp.minimum(prefix + carry, POS_VOCAB - PWIN)
                st = pl.multiple_of((st // 8) * 8, 8)
                starts.append(st)
                dels.append(prefix + carry - st)
                cnt_sub = jnp.zeros((), jnp.int32)
            m = mask_v[pl.ds(base + k * L, L)]
            mq = m == 1
            plsc.store_compressed(crow_v.at[pl.ds(s * SUB + cnt_sub, L)],
                                  kk * L + lane, mask=mq)
            nm = jnp.sum(jnp.where(mq, 1, 0))
            cnt_sub = cnt_sub + nm
            carry = carry + nm
            if kk == SUB // L - 1:
                counts.append(cnt_sub)

    gp = pltpu.async_copy(ptab_hbm.at[pl.ds(starts[0], PWIN)], pbuf, sem_p)

    for s in range(NSUB):
        with jax.named_scope("nk_wait"):
            for j in range(SPS):
                gn[SPS * s + j].wait()
            gp.wait()
        with jax.named_scope("nk_add"):
            def add_body(i, _, nb=nbufs[s], d=dels[s], s=s):
                j = crow_v[pl.ds(s * SUB + i, L)][0]
                rel = d + i
                for cb in range(HID // L):
                    nb[j, pl.ds(cb * L, L)] = (
                        nb[j, pl.ds(cb * L, L)]
                        + pbuf[rel, pl.ds(cb * L, L)])
                return 0

            lax.fori_loop(0, counts[s], add_body, 0)
        pltpu.async_copy(nbufs[s], out_hbm.at[pl.ds(base + s * SUB, SUB)],
                         sems_w[s])
        if s + 1 < NSUB:
            gp = pltpu.async_copy(ptab_hbm.at[pl.ds(starts[s + 1], PWIN)],
                                  pbuf, sem_p)
    with jax.named_scope("nk_drain"):
        for s in range(NSUB):
            pltpu.make_async_copy(
                nbufs[s], out_hbm.at[pl.ds(base + s * SUB, SUB)],
                sems_w[s]).wait()


def _edge_body(ids_ref, tbl_ref, g_ref, b_ref, out_ref):
    tbl = tbl_ref[...]
    mean = jnp.mean(tbl, axis=1, keepdims=True)
    var = jnp.mean((tbl - mean) ** 2, axis=1, keepdims=True)
    norm = (tbl - mean) * lax.rsqrt(var + 1e-5) * g_ref[...] + b_ref[...]
    norm_hi = norm.astype(jnp.bfloat16)
    norm_lo = (norm - norm_hi.astype(jnp.float32)).astype(jnp.bfloat16)
    ids = ids_ref[0, 0, :]
    onehot = (ids[:, None] == lax.broadcasted_iota(jnp.int32, (BE, 64), 1)
              ).astype(jnp.bfloat16)
    out_ref[...] = (
        jnp.dot(onehot, norm_hi, preferred_element_type=jnp.float32)
        + jnp.dot(onehot, norm_lo, preferred_element_type=jnp.float32))


@jax.jit
def kernel(node_ids, top_mask, edge_ids, node_table, pos_table, edge_table,
           ln_gamma, ln_beta):
    node_ids = node_ids.astype(jnp.int32)
    top_mask = top_mask.astype(jnp.int32)

    node_feat = pl.kernel(
        _node_body,
        out_type=jax.ShapeDtypeStruct((N_NODES, HID), jnp.float32),
        mesh=plsc.VectorSubcoreMesh(core_axis_name="c", subcore_axis_name="s",
                                    num_cores=NC, num_subcores=NS),
        scratch_types=[
            pltpu.VMEM((C,), jnp.int32),
            pltpu.VMEM((N_NODES + L,), jnp.int32),
            pltpu.VMEM((C + L,), jnp.int32),
            [pltpu.VMEM((SUB, HID), jnp.float32)] * NSUB,
            pltpu.VMEM((PWIN, HID), jnp.float32),
            [pltpu.SemaphoreType.DMA] * NSUB,
            pltpu.SemaphoreType.DMA,
            [pltpu.SemaphoreType.DMA] * NSUB,
            pltpu.SemaphoreType.DMA,
        ],
        compiler_params=pltpu.CompilerParams(needs_layout_passes=False),
    )(node_ids, top_mask, node_table, pos_table)

    edge_feat = pl.pallas_call(
        _edge_body,
        grid=(NB,),
        in_specs=[
            pl.BlockSpec((1, 1, BE), lambda i: (i, 0, 0)),
            pl.BlockSpec((64, HID), lambda i: (0, 0)),
            pl.BlockSpec((1, HID), lambda i: (0, 0)),
            pl.BlockSpec((1, HID), lambda i: (0, 0)),
        ],
        out_specs=pl.BlockSpec((BE, HID), lambda i: (i, 0)),
        out_shape=jax.ShapeDtypeStruct((N_EDGES, HID), jnp.float32),
    )(edge_ids.reshape(NB, 1, BE), edge_table,
      ln_gamma.reshape(1, HID), ln_beta.reshape(1, HID))

    return node_feat, edge_feat
